# Initial kernel scaffold; baseline (speedup 1.0000x reference)
import jax
import jax.numpy as jnp
from jax.experimental import pallas as pl


def kernel(x, edge_index, batch, W1, b1, W2, b2, W3, b3, W4, b4, bn_gamma, bn_beta, fc1_W, fc1_b, fc2_W, fc2_b, fc3_W, fc3_b):
    raise NotImplementedError("write your pallas kernel here")


# speedup vs baseline: 7.5295x; 7.5295x over previous
import functools

import jax
import jax.numpy as jnp
from jax import lax
from jax.experimental import pallas as pl
from jax.experimental.pallas import tpu as pltpu
from jax.experimental.pallas import tpu_sc as plsc

_N = 10000
_E = 320000
_G = 32
_NCLS = 2

_NTILES = 16
_NCORES = 2
_NW = _NTILES * _NCORES
_CHUNK = 128
_CPW = 80
_E_PAD = _NW * _CPW * _CHUNK
_N_PAD = 10240
_RPT = _N_PAD // _NTILES


@functools.lru_cache(maxsize=None)
def _make_agg(w):
    mesh = plsc.VectorSubcoreMesh(core_axis_name="c", subcore_axis_name="s")

    @functools.partial(
        pl.kernel,
        mesh=mesh,
        compiler_params=pltpu.CompilerParams(use_tc_tiling_on_sc=False),
        out_type=jax.ShapeDtypeStruct((_NCORES, _N_PAD, w), jnp.float32),
        scratch_types=[
            pltpu.VMEM((_CPW, _CHUNK), jnp.int32),
            pltpu.VMEM((_CPW, _CHUNK), jnp.int32),
            pltpu.VMEM((_CHUNK, w), jnp.float32),
            pltpu.VMEM((_RPT, w), jnp.float32),
            pltpu.VMEM_SHARED((_N_PAD, w), jnp.float32),
            pltpu.SemaphoreType.DMA,
        ],
    )
    def agg(table_hbm, edges_hbm, zeros_hbm, out_hbm,
            src_v, dst_v, rows_v, buf_v, acc_sh, sem):
        c = lax.axis_index("c")
        s = lax.axis_index("s")
        wid = c * _NTILES + s
        pltpu.sync_copy(zeros_hbm, buf_v)
        pltpu.sync_copy(buf_v, acc_sh.at[pl.ds(s * _RPT, _RPT)])
        pltpu.sync_copy(edges_hbm.at[0].at[pl.ds(wid * _CPW, _CPW)], src_v)
        pltpu.sync_copy(edges_hbm.at[1].at[pl.ds(wid * _CPW, _CPW)], dst_v)
        plsc.subcore_barrier()

        def body(j, carry):
            pltpu.async_copy(table_hbm.at[src_v.at[j]], rows_v, sem).wait()
            pltpu.sync_copy(rows_v, acc_sh.at[dst_v.at[j]], add=True)
            return carry

        lax.fori_loop(0, _CPW, body, 0)
        plsc.subcore_barrier()
        pltpu.sync_copy(acc_sh.at[pl.ds(s * _RPT, _RPT)], buf_v)
        pltpu.sync_copy(buf_v, out_hbm.at[c].at[pl.ds(s * _RPT, _RPT)])

    return agg


def _lrelu(v):
    return jnp.where(v >= 0, v, 0.01 * v)


def _tc_matmul(x, W):
    def body(x_ref, w_ref, o_ref):
        o_ref[...] = jnp.dot(x_ref[...], w_ref[...],
                             preferred_element_type=jnp.float32)

    return pl.pallas_call(
        body,
        out_shape=jax.ShapeDtypeStruct((x.shape[0], W.shape[1]), jnp.float32),
    )(x, W)


def _tc_bias_lrelu(P, b):
    def body(p_ref, b_ref, o_ref):
        v = p_ref[0] + p_ref[1] + b_ref[...]
        o_ref[...] = _lrelu(v)

    return pl.pallas_call(
        body,
        out_shape=jax.ShapeDtypeStruct(P.shape[1:], jnp.float32),
    )(P, b.reshape(1, -1))


def _tc_mm_bias_lrelu(P, W, b):
    def body(p_ref, w_ref, b_ref, o_ref):
        agg = p_ref[0] + p_ref[1]
        v = jnp.dot(agg, w_ref[...], preferred_element_type=jnp.float32)
        o_ref[...] = _lrelu(v + b_ref[...])

    return pl.pallas_call(
        body,
        out_shape=jax.ShapeDtypeStruct((P.shape[1], W.shape[1]), jnp.float32),
    )(P, W, b.reshape(1, -1))


def _tc_head(P, W4, b4, gamma, beta, batch_pad):
    def body(p_ref, w_ref, b_ref, g_ref, be_ref, batch_ref,
             f1w_ref, f1b_ref, f2w_ref, f2b_ref, f3w_ref, f3b_ref, o_ref):
        agg = p_ref[0] + p_ref[1]
        h = jnp.dot(agg, w_ref[...], preferred_element_type=jnp.float32)
        h = h + b_ref[...]
        rows = lax.broadcasted_iota(jnp.int32, (_N_PAD, 1), 0)
        mask = (rows < _N).astype(jnp.float32)
        hm = h * mask
        s1 = jnp.sum(hm, axis=0, keepdims=True)
        s2 = jnp.sum(hm * hm, axis=0, keepdims=True)
        mean = s1 / _N
        var = s2 / _N - mean * mean
        hn = (h - mean) * lax.rsqrt(var + 1e-5) * g_ref[...] + be_ref[...]
        hn = _lrelu(hn)
        gids = lax.broadcasted_iota(jnp.int32, (1, _G), 1)
        onehot = (batch_ref[...] == gids).astype(jnp.float32)
        pooled = lax.dot_general(onehot, hn, (((0,), (0,)), ((), ())),
                                 preferred_element_type=jnp.float32)
        o = _lrelu(jnp.dot(pooled, f1w_ref[...],
                           preferred_element_type=jnp.float32) + f1b_ref[...])
        o = _lrelu(jnp.dot(o, f2w_ref[...],
                           preferred_element_type=jnp.float32) + f2b_ref[...])
        o = jnp.dot(o, f3w_ref[...],
                    preferred_element_type=jnp.float32) + f3b_ref[...]
        m = jnp.max(o, axis=-1, keepdims=True)
        lse = m + jnp.log(jnp.sum(jnp.exp(o - m), axis=-1, keepdims=True))
        o_ref[...] = o - lse

    return pl.pallas_call(
        body,
        out_shape=jax.ShapeDtypeStruct((_G, _NCLS), jnp.float32),
    )


def kernel(x, edge_index, batch, W1, b1, W2, b2, W3, b3, W4, b4,
           bn_gamma, bn_beta, fc1_W, fc1_b, fc2_W, fc2_b, fc3_W, fc3_b):
    pad_e = _E_PAD - _E
    src = jnp.concatenate([edge_index[0], jnp.zeros((pad_e,), jnp.int32)])
    dst = jnp.concatenate([edge_index[1], jnp.full((pad_e,), _N, jnp.int32)])
    edges = jnp.stack([src, dst]).reshape(2, _NW * _CPW, _CHUNK)
    x_pad = jnp.pad(x, ((0, _N_PAD - _N), (0, 0)))
    batch_pad = jnp.concatenate(
        [batch, jnp.full((_N_PAD - _N,), _G, jnp.int32)]).reshape(_N_PAD, 1)
    zeros = {w: jnp.zeros((_RPT, w), jnp.float32) for w in (16, 32, 64)}

    m1 = _tc_matmul(x_pad, W1)
    P = _make_agg(16)(m1, edges, zeros[16])
    h1 = _tc_bias_lrelu(P, b1)
    P = _make_agg(16)(h1, edges, zeros[16])
    h2 = _tc_mm_bias_lrelu(P, W2, b2)
    P = _make_agg(32)(h2, edges, zeros[32])
    h3 = _tc_mm_bias_lrelu(P, W3, b3)
    P = _make_agg(64)(h3, edges, zeros[64])
    out = _tc_head(P, W4, b4, bn_gamma, bn_beta, batch_pad)(
        P, W4, b4.reshape(1, -1), bn_gamma.reshape(1, -1),
        bn_beta.reshape(1, -1), batch_pad,
        fc1_W, fc1_b.reshape(1, -1), fc2_W, fc2_b.reshape(1, -1),
        fc3_W, fc3_b.reshape(1, -1))
    return out


# speedup vs baseline: 13.8360x; 1.8376x over previous
import functools

import jax
import jax.numpy as jnp
from jax import lax
from jax.experimental import pallas as pl
from jax.experimental.pallas import tpu as pltpu
from jax.experimental.pallas import tpu_sc as plsc

_N = 10000
_E = 320000
_G = 32
_NCLS = 2

_NTILES = 16
_NCORES = 2
_NW = _NTILES * _NCORES
_CHUNK = 128
_CPW = 80
_E_PAD = _NW * _CPW * _CHUNK
_N_PAD = 10240
_RPT = _N_PAD // _NTILES


@functools.lru_cache(maxsize=None)
def _make_agg(w):
    mesh = plsc.VectorSubcoreMesh(core_axis_name="c", subcore_axis_name="s")

    @functools.partial(
        pl.kernel,
        mesh=mesh,
        compiler_params=pltpu.CompilerParams(use_tc_tiling_on_sc=False),
        out_type=jax.ShapeDtypeStruct((_NCORES, _N_PAD, w), jnp.float32),
        scratch_types=[
            pltpu.VMEM((_CPW, _CHUNK), jnp.int32),
            pltpu.VMEM((_CPW, _CHUNK), jnp.int32),
            pltpu.VMEM((_CHUNK, w), jnp.float32),
            pltpu.VMEM((_CHUNK, w), jnp.float32),
            pltpu.VMEM((_RPT, w), jnp.float32),
            pltpu.VMEM_SHARED((_N_PAD, w), jnp.float32),
            pltpu.SemaphoreType.DMA,
            pltpu.SemaphoreType.DMA,
            pltpu.SemaphoreType.DMA,
            pltpu.SemaphoreType.DMA,
        ],
    )
    def agg(table_hbm, edges_hbm, zeros_hbm, out_hbm,
            src_v, dst_v, rows_a, rows_b, buf_v, acc_sh,
            ga, gb, sa, sb):
        c = lax.axis_index("c")
        s = lax.axis_index("s")
        wid = c * _NTILES + s
        pltpu.sync_copy(zeros_hbm, buf_v)
        pltpu.sync_copy(buf_v, acc_sh.at[pl.ds(s * _RPT, _RPT)])
        pltpu.sync_copy(edges_hbm.at[0].at[pl.ds(wid * _CPW, _CPW)], src_v)
        pltpu.sync_copy(edges_hbm.at[1].at[pl.ds(wid * _CPW, _CPW)], dst_v)
        plsc.subcore_barrier()

        def gather(j, buf, sem):
            pltpu.async_copy(table_hbm.at[src_v.at[j]], buf, sem)

        def gather_wait(buf, sem):
            pltpu.make_async_copy(table_hbm.at[src_v.at[0]], buf, sem).wait()

        def scat_wait(buf, sem):
            pltpu.make_async_copy(buf, acc_sh.at[dst_v.at[0]], sem).wait()

        gather(0, rows_a, ga)
        gather(1, rows_b, gb)

        def body(k, carry):
            j0 = 2 * k
            gather_wait(rows_a, ga)
            pltpu.async_copy(rows_a, acc_sh.at[dst_v.at[j0]], sa, add=True)
            gather_wait(rows_b, gb)
            pltpu.async_copy(rows_b, acc_sh.at[dst_v.at[j0 + 1]], sb, add=True)
            scat_wait(rows_a, sa)
            gather(jnp.minimum(j0 + 2, _CPW - 2), rows_a, ga)
            scat_wait(rows_b, sb)
            gather(jnp.minimum(j0 + 3, _CPW - 1), rows_b, gb)
            return carry

        lax.fori_loop(0, _CPW // 2, body, 0)
        gather_wait(rows_a, ga)
        gather_wait(rows_b, gb)
        plsc.subcore_barrier()
        pltpu.sync_copy(acc_sh.at[pl.ds(s * _RPT, _RPT)], buf_v)
        pltpu.sync_copy(buf_v, out_hbm.at[c].at[pl.ds(s * _RPT, _RPT)])

    return agg


def _lrelu(v):
    return jnp.where(v >= 0, v, 0.01 * v)


def _dot(a, b):
    return jnp.dot(a, b, preferred_element_type=jnp.float32,
                   precision=lax.Precision.HIGHEST)


def _tc_matmul(x, W):
    def body(x_ref, w_ref, o_ref):
        o_ref[...] = _dot(x_ref[...], w_ref[...])

    return pl.pallas_call(
        body,
        out_shape=jax.ShapeDtypeStruct((x.shape[0], W.shape[1]), jnp.float32),
    )(x, W)


def _tc_bias_lrelu(P, b):
    def body(p_ref, b_ref, o_ref):
        v = p_ref[0] + p_ref[1] + b_ref[...]
        o_ref[...] = _lrelu(v)

    return pl.pallas_call(
        body,
        out_shape=jax.ShapeDtypeStruct(P.shape[1:], jnp.float32),
    )(P, b.reshape(1, -1))


def _tc_mm_bias_lrelu(P, W, b):
    def body(p_ref, w_ref, b_ref, o_ref):
        agg = p_ref[0] + p_ref[1]
        v = _dot(agg, w_ref[...])
        o_ref[...] = _lrelu(v + b_ref[...])

    return pl.pallas_call(
        body,
        out_shape=jax.ShapeDtypeStruct((P.shape[1], W.shape[1]), jnp.float32),
    )(P, W, b.reshape(1, -1))


def _tc_head(P, W4, b4, gamma, beta, batch_pad):
    def body(p_ref, w_ref, b_ref, g_ref, be_ref, batch_ref,
             f1w_ref, f1b_ref, f2w_ref, f2b_ref, f3w_ref, f3b_ref, o_ref):
        agg = p_ref[0] + p_ref[1]
        h = _dot(agg, w_ref[...])
        h = h + b_ref[...]
        rows = lax.broadcasted_iota(jnp.int32, (_N_PAD, 1), 0)
        mask = (rows < _N).astype(jnp.float32)
        hm = h * mask
        s1 = jnp.sum(hm, axis=0, keepdims=True)
        mean = s1 / _N
        diff = (h - mean) * mask
        var = jnp.sum(diff * diff, axis=0, keepdims=True) / _N
        hn = (h - mean) * lax.rsqrt(var + 1e-5) * g_ref[...] + be_ref[...]
        hn = _lrelu(hn)
        gids = lax.broadcasted_iota(jnp.int32, (1, _G), 1)
        onehot = (batch_ref[...] == gids).astype(jnp.float32)
        pooled = lax.dot_general(onehot, hn, (((0,), (0,)), ((), ())),
                                 preferred_element_type=jnp.float32,
                                 precision=lax.Precision.HIGHEST)
        o = _lrelu(_dot(pooled, f1w_ref[...]) + f1b_ref[...])
        o = _lrelu(_dot(o, f2w_ref[...]) + f2b_ref[...])
        o = _dot(o, f3w_ref[...]) + f3b_ref[...]
        m = jnp.max(o, axis=-1, keepdims=True)
        lse = m + jnp.log(jnp.sum(jnp.exp(o - m), axis=-1, keepdims=True))
        o_ref[...] = o - lse

    return pl.pallas_call(
        body,
        out_shape=jax.ShapeDtypeStruct((_G, _NCLS), jnp.float32),
    )


def kernel(x, edge_index, batch, W1, b1, W2, b2, W3, b3, W4, b4,
           bn_gamma, bn_beta, fc1_W, fc1_b, fc2_W, fc2_b, fc3_W, fc3_b):
    pad_e = _E_PAD - _E
    pad_ids = jnp.arange(pad_e, dtype=jnp.int32)
    src = jnp.concatenate([edge_index[0], (pad_ids * 37) % _N])
    dst = jnp.concatenate([edge_index[1], _N + (pad_ids % (_N_PAD - _N))])
    edges = jnp.stack([src, dst]).reshape(2, _NW * _CPW, _CHUNK)
    x_pad = jnp.pad(x, ((0, _N_PAD - _N), (0, 0)))
    batch_pad = jnp.concatenate(
        [batch, jnp.full((_N_PAD - _N,), _G, jnp.int32)]).reshape(_N_PAD, 1)
    zeros = {w: jnp.zeros((_RPT, w), jnp.float32) for w in (16, 32, 64)}

    m1 = _tc_matmul(x_pad, W1)
    P = _make_agg(16)(m1, edges, zeros[16])
    h1 = _tc_bias_lrelu(P, b1)
    P = _make_agg(16)(h1, edges, zeros[16])
    h2 = _tc_mm_bias_lrelu(P, W2, b2)
    P = _make_agg(32)(h2, edges, zeros[32])
    h3 = _tc_mm_bias_lrelu(P, W3, b3)
    P = _make_agg(64)(h3, edges, zeros[64])
    out = _tc_head(P, W4, b4, bn_gamma, bn_beta, batch_pad)(
        P, W4, b4.reshape(1, -1), bn_gamma.reshape(1, -1),
        bn_beta.reshape(1, -1), batch_pad,
        fc1_W, fc1_b.reshape(1, -1), fc2_W, fc2_b.reshape(1, -1),
        fc3_W, fc3_b.reshape(1, -1))
    return out


# speedup vs baseline: 15.7847x; 1.1408x over previous
import functools

import jax
import jax.numpy as jnp
from jax import lax
from jax.experimental import pallas as pl
from jax.experimental.pallas import tpu as pltpu
from jax.experimental.pallas import tpu_sc as plsc

_N = 10000
_E = 320000
_G = 32
_NCLS = 2

_NTILES = 16
_NCORES = 2
_NW = _NTILES * _NCORES
_CHUNK = 256
_CPW = 40
_E_PAD = _NW * _CPW * _CHUNK
_N_PAD = 10240
_RPT = _N_PAD // _NTILES


@functools.lru_cache(maxsize=None)
def _make_agg(w):
    mesh = plsc.VectorSubcoreMesh(core_axis_name="c", subcore_axis_name="s")

    @functools.partial(
        pl.kernel,
        mesh=mesh,
        compiler_params=pltpu.CompilerParams(use_tc_tiling_on_sc=False),
        out_type=jax.ShapeDtypeStruct((_NCORES, _N_PAD, w), jnp.float32),
        scratch_types=[
            pltpu.VMEM((_CPW, _CHUNK), jnp.int32),
            pltpu.VMEM((_CPW, _CHUNK), jnp.int32),
            pltpu.VMEM((_CHUNK, w), jnp.float32),
            pltpu.VMEM((_CHUNK, w), jnp.float32),
            pltpu.VMEM_SHARED((_N_PAD, w), jnp.float32),
            pltpu.SemaphoreType.DMA,
            pltpu.SemaphoreType.DMA,
            pltpu.SemaphoreType.DMA,
            pltpu.SemaphoreType.DMA,
        ],
    )
    def agg(table_hbm, edges_hbm, zeros_hbm, out_hbm,
            src_v, dst_v, rows_a, rows_b, acc_sh,
            ga, gb, sa, sb):
        c = lax.axis_index("c")
        s = lax.axis_index("s")
        wid = c * _NTILES + s
        pltpu.sync_copy(zeros_hbm, acc_sh.at[pl.ds(s * _RPT, _RPT)])
        pltpu.sync_copy(edges_hbm.at[0].at[pl.ds(wid * _CPW, _CPW)], src_v)
        pltpu.sync_copy(edges_hbm.at[1].at[pl.ds(wid * _CPW, _CPW)], dst_v)
        plsc.subcore_barrier()

        def gather(j, buf, sem):
            pltpu.async_copy(table_hbm.at[src_v.at[j]], buf, sem)

        def gather_wait(buf, sem):
            pltpu.make_async_copy(table_hbm.at[src_v.at[0]], buf, sem).wait()

        def scat_wait(buf, sem):
            pltpu.make_async_copy(buf, acc_sh.at[dst_v.at[0]], sem).wait()

        gather(0, rows_a, ga)
        gather(1, rows_b, gb)

        def body(k, carry):
            j0 = 2 * k
            gather_wait(rows_a, ga)
            pltpu.async_copy(rows_a, acc_sh.at[dst_v.at[j0]], sa, add=True)
            gather_wait(rows_b, gb)
            pltpu.async_copy(rows_b, acc_sh.at[dst_v.at[j0 + 1]], sb, add=True)
            scat_wait(rows_a, sa)
            gather(jnp.minimum(j0 + 2, _CPW - 2), rows_a, ga)
            scat_wait(rows_b, sb)
            gather(jnp.minimum(j0 + 3, _CPW - 1), rows_b, gb)
            return carry

        lax.fori_loop(0, _CPW // 2, body, 0)
        gather_wait(rows_a, ga)
        gather_wait(rows_b, gb)
        plsc.subcore_barrier()
        pltpu.sync_copy(acc_sh.at[pl.ds(s * _RPT, _RPT)],
                        out_hbm.at[c].at[pl.ds(s * _RPT, _RPT)])

    return agg


def _lrelu(v):
    return jnp.where(v >= 0, v, 0.01 * v)


def _dot(a, b):
    return jnp.dot(a, b, preferred_element_type=jnp.float32,
                   precision=lax.Precision.HIGHEST)


def _tc_matmul(x, W):
    def body(x_ref, w_ref, o_ref):
        o_ref[...] = _dot(x_ref[...], w_ref[...])

    return pl.pallas_call(
        body,
        out_shape=jax.ShapeDtypeStruct((x.shape[0], W.shape[1]), jnp.float32),
    )(x, W)


def _tc_bias_lrelu(P, b):
    def body(p_ref, b_ref, o_ref):
        v = p_ref[0] + p_ref[1] + b_ref[...]
        o_ref[...] = _lrelu(v)

    return pl.pallas_call(
        body,
        out_shape=jax.ShapeDtypeStruct(P.shape[1:], jnp.float32),
    )(P, b.reshape(1, -1))


def _tc_mm_bias_lrelu(P, W, b):
    def body(p_ref, w_ref, b_ref, o_ref):
        agg = p_ref[0] + p_ref[1]
        v = _dot(agg, w_ref[...])
        o_ref[...] = _lrelu(v + b_ref[...])

    return pl.pallas_call(
        body,
        out_shape=jax.ShapeDtypeStruct((P.shape[1], W.shape[1]), jnp.float32),
    )(P, W, b.reshape(1, -1))


def _tc_head(P, W4, b4, gamma, beta, batch_pad):
    def body(p_ref, w_ref, b_ref, g_ref, be_ref, batch_ref,
             f1w_ref, f1b_ref, f2w_ref, f2b_ref, f3w_ref, f3b_ref, o_ref):
        agg = p_ref[0] + p_ref[1]
        h = _dot(agg, w_ref[...])
        h = h + b_ref[...]
        rows = lax.broadcasted_iota(jnp.int32, (_N_PAD, 1), 0)
        mask = (rows < _N).astype(jnp.float32)
        hm = h * mask
        s1 = jnp.sum(hm, axis=0, keepdims=True)
        mean = s1 / _N
        diff = (h - mean) * mask
        var = jnp.sum(diff * diff, axis=0, keepdims=True) / _N
        hn = (h - mean) * lax.rsqrt(var + 1e-5) * g_ref[...] + be_ref[...]
        hn = _lrelu(hn)
        gids = lax.broadcasted_iota(jnp.int32, (1, _G), 1)
        onehot = (batch_ref[...] == gids).astype(jnp.float32)
        pooled = lax.dot_general(onehot, hn, (((0,), (0,)), ((), ())),
                                 preferred_element_type=jnp.float32,
                                 precision=lax.Precision.HIGHEST)
        o = _lrelu(_dot(pooled, f1w_ref[...]) + f1b_ref[...])
        o = _lrelu(_dot(o, f2w_ref[...]) + f2b_ref[...])
        o = _dot(o, f3w_ref[...]) + f3b_ref[...]
        m = jnp.max(o, axis=-1, keepdims=True)
        lse = m + jnp.log(jnp.sum(jnp.exp(o - m), axis=-1, keepdims=True))
        o_ref[...] = o - lse

    return pl.pallas_call(
        body,
        out_shape=jax.ShapeDtypeStruct((_G, _NCLS), jnp.float32),
    )


def kernel(x, edge_index, batch, W1, b1, W2, b2, W3, b3, W4, b4,
           bn_gamma, bn_beta, fc1_W, fc1_b, fc2_W, fc2_b, fc3_W, fc3_b):
    pad_e = _E_PAD - _E
    pad_ids = jnp.arange(pad_e, dtype=jnp.int32)
    src = jnp.concatenate([edge_index[0], (pad_ids * 37) % _N])
    dst = jnp.concatenate([edge_index[1], _N + (pad_ids % (_N_PAD - _N))])
    edges = jnp.stack([src, dst]).reshape(2, _NW * _CPW, _CHUNK)
    x_pad = jnp.pad(x, ((0, _N_PAD - _N), (0, 0)))
    batch_pad = jnp.concatenate(
        [batch, jnp.full((_N_PAD - _N,), _G, jnp.int32)]).reshape(_N_PAD, 1)
    zeros = {w: jnp.zeros((_RPT, w), jnp.float32) for w in (16, 32, 64)}

    m1 = _tc_matmul(x_pad, W1)
    P = _make_agg(16)(m1, edges, zeros[16])
    h1 = _tc_bias_lrelu(P, b1)
    P = _make_agg(16)(h1, edges, zeros[16])
    h2 = _tc_mm_bias_lrelu(P, W2, b2)
    P = _make_agg(32)(h2, edges, zeros[32])
    h3 = _tc_mm_bias_lrelu(P, W3, b3)
    P = _make_agg(64)(h3, edges, zeros[64])
    out = _tc_head(P, W4, b4, bn_gamma, bn_beta, batch_pad)(
        P, W4, b4.reshape(1, -1), bn_gamma.reshape(1, -1),
        bn_beta.reshape(1, -1), batch_pad,
        fc1_W, fc1_b.reshape(1, -1), fc2_W, fc2_b.reshape(1, -1),
        fc3_W, fc3_b.reshape(1, -1))
    return out
